# baseline (speedup 1.0000x reference)
import jax
import jax.numpy as jnp
from jax import lax
from jax.experimental import pallas as pl
from jax.experimental.pallas import tpu as pltpu
from jax.experimental.pallas import tpu_sc as plsc

NC = 2
NS = 16
NW = NC * NS
D = 128
LANES = 16
NCH = D // LANES
CH = 16
HALF = 104
NNEG_PAD = 2 * HALF
MARGIN = 0.5


def _sc_body(uids, pids, nids, utab, itab, out,
             uidv, pidv, nidv, urows, prows, nrows, lossv,
             semn0, semn1, semu, semp):
    wid = lax.axis_index("s") * NC + lax.axis_index("c")
    bt = uids.shape[0] // NW
    n_chunks = bt // CH
    base = wid * bt

    def stage_chunk(lc, q):
        b0 = base + lc * CH
        pltpu.sync_copy(uids.at[pl.ds(b0, CH)], uidv.at[q])
        pltpu.sync_copy(pids.at[pl.ds(b0, CH)], pidv.at[q])
        pltpu.sync_copy(nids.at[pl.ds(b0, CH)], nidv.at[q])
        pltpu.async_copy(utab.at[uidv.at[q]], urows.at[q], semu)
        pltpu.async_copy(itab.at[pidv.at[q]], prows.at[q], semp)

    def issue_neg(bl, p):
        q = (bl // CH) & 1
        bi = bl % CH
        pltpu.async_copy(itab.at[nidv.at[q, bi, 0]],
                         nrows.at[p, pl.ds(0, HALF)], semn0)
        pltpu.async_copy(itab.at[nidv.at[q, bi, 1]],
                         nrows.at[p, pl.ds(HALF, HALF)], semn0)

    stage_chunk(0, 0)
    issue_neg(0, 0)

    def b_iter(bl, tl):
        p = bl & 1
        lc = bl // CH
        q = lc & 1
        bi = bl % CH

        @pl.when(bi == 0)
        def _():
            pltpu.make_async_copy(utab.at[pl.ds(0, CH)],
                                  urows.at[q], semu).wait()
            pltpu.make_async_copy(itab.at[pl.ds(0, CH)],
                                  prows.at[q], semp).wait()

        pltpu.make_async_copy(itab.at[pl.ds(0, NNEG_PAD)],
                              nrows.at[p], semn0).wait()

        @pl.when((bi == CH - 1) & (lc + 1 < n_chunks))
        def _():
            stage_chunk(lc + 1, 1 - q)

        @pl.when(bl + 1 < bt)
        def _():
            issue_neg(bl + 1, 1 - p)

        u = [urows[q, bi, pl.ds(c * LANES, LANES)] for c in range(NCH)]
        accp = jnp.zeros((LANES,), jnp.float32)
        for c in range(NCH):
            dv = prows[q, bi, pl.ds(c * LANES, LANES)] - u[c]
            accp = accp + dv * dv
        pd = jnp.sum(accp)

        def neg_row(n, mn):
            acc = jnp.zeros((LANES,), jnp.float32)
            for c in range(NCH):
                dv = nrows[p, n, pl.ds(c * LANES, LANES)] - u[c]
                acc = acc + dv * dv
            return jnp.minimum(mn, jnp.sum(acc))

        mn = plsc.parallel_loop(0, NNEG_PAD, 1, unroll=4,
                                carry=jnp.float32(3.0e38))(neg_row)
        delta = mn - pd
        return tl + jnp.abs(delta - jnp.float32(MARGIN))

    tile_loss = lax.fori_loop(0, bt, b_iter, jnp.float32(0.0))
    lossv[...] = jnp.broadcast_to(tile_loss, (LANES,))
    pltpu.sync_copy(lossv, out.at[wid])


def kernel(user_ids, pos_ids, neg_ids, user_table, item_table):
    batch, nneg = neg_ids.shape
    nid2 = jnp.concatenate(
        [neg_ids[:, :100], neg_ids[:, :4], neg_ids[:, 100:], neg_ids[:, 4:8]],
        axis=1).reshape(batch, 2, HALF)
    mesh = plsc.VectorSubcoreMesh(core_axis_name="c", subcore_axis_name="s")
    f = pl.kernel(
        _sc_body,
        out_type=jax.ShapeDtypeStruct((NW, LANES), jnp.float32),
        mesh=mesh,
        compiler_params=pltpu.CompilerParams(needs_layout_passes=False),
        scratch_types=[
            pltpu.VMEM((2, CH), jnp.int32),
            pltpu.VMEM((2, CH), jnp.int32),
            pltpu.VMEM((2, CH, 2, HALF), jnp.int32),
            pltpu.VMEM((2, CH, D), jnp.float32),
            pltpu.VMEM((2, CH, D), jnp.float32),
            pltpu.VMEM((2, NNEG_PAD, D), jnp.float32),
            pltpu.VMEM((LANES,), jnp.float32),
            pltpu.SemaphoreType.DMA,
            pltpu.SemaphoreType.DMA,
            pltpu.SemaphoreType.DMA,
            pltpu.SemaphoreType.DMA,
        ],
    )
    partials = f(user_ids, pos_ids, nid2, user_table, item_table)
    return jnp.sum(partials[:, 0])


# speedup vs baseline: 1.0722x; 1.0722x over previous
import jax
import jax.numpy as jnp
from jax import lax
from jax.experimental import pallas as pl
from jax.experimental.pallas import tpu as pltpu
from jax.experimental.pallas import tpu_sc as plsc

NC = 2
NS = 16
NW = NC * NS
D = 128
LANES = 16
NCH = D // LANES
CH = 32
HALF = 104
NNEG_PAD = 2 * HALF
MARGIN = 0.5


def _sc_body(uids, pids, nids, utab, itab, out,
             uidv, pidv, nidv, urows, prows, nrows, lossv,
             semn0, semn1, semu, semp):
    wid = lax.axis_index("s") * NC + lax.axis_index("c")
    bt = uids.shape[0] // NW
    n_chunks = bt // CH
    base = wid * bt

    def stage_chunk(lc, q):
        b0 = base + lc * CH
        pltpu.sync_copy(uids.at[pl.ds(b0, CH)], uidv.at[q])
        pltpu.sync_copy(pids.at[pl.ds(b0, CH)], pidv.at[q])
        pltpu.sync_copy(nids.at[pl.ds(b0, CH)], nidv.at[q])
        pltpu.async_copy(utab.at[uidv.at[q]], urows.at[q], semu)
        pltpu.async_copy(itab.at[pidv.at[q]], prows.at[q], semp)

    def issue_neg(bl, p):
        q = (bl // CH) & 1
        bi = bl % CH
        pltpu.async_copy(itab.at[nidv.at[q, bi, 0]],
                         nrows.at[p, pl.ds(0, HALF)], semn0)
        pltpu.async_copy(itab.at[nidv.at[q, bi, 1]],
                         nrows.at[p, pl.ds(HALF, HALF)], semn1)

    stage_chunk(0, 0)
    issue_neg(0, 0)

    def b_iter(bl, tl):
        p = bl & 1
        lc = bl // CH
        q = lc & 1
        bi = bl % CH

        @pl.when(bi == 0)
        def _():
            pltpu.make_async_copy(utab.at[pl.ds(0, CH)],
                                  urows.at[q], semu).wait()
            pltpu.make_async_copy(itab.at[pl.ds(0, CH)],
                                  prows.at[q], semp).wait()

        pltpu.make_async_copy(itab.at[pl.ds(0, HALF)],
                              nrows.at[p, pl.ds(0, HALF)], semn0).wait()
        pltpu.make_async_copy(itab.at[pl.ds(0, HALF)],
                              nrows.at[p, pl.ds(HALF, HALF)], semn1).wait()

        @pl.when((bi == CH - 1) & (lc + 1 < n_chunks))
        def _():
            stage_chunk(lc + 1, 1 - q)

        @pl.when(bl + 1 < bt)
        def _():
            issue_neg(bl + 1, 1 - p)

        u = [urows[q, bi, pl.ds(c * LANES, LANES)] for c in range(NCH)]
        accp = jnp.zeros((LANES,), jnp.float32)
        for c in range(NCH):
            dv = prows[q, bi, pl.ds(c * LANES, LANES)] - u[c]
            accp = accp + dv * dv
        pd = jnp.sum(accp)

        def neg_row(n, mn):
            acc = jnp.zeros((LANES,), jnp.float32)
            for c in range(NCH):
                dv = nrows[p, n, pl.ds(c * LANES, LANES)] - u[c]
                acc = acc + dv * dv
            return jnp.minimum(mn, jnp.sum(acc))

        mn = plsc.parallel_loop(0, NNEG_PAD, 1, unroll=4,
                                carry=jnp.float32(3.0e38))(neg_row)
        delta = mn - pd
        return tl + jnp.abs(delta - jnp.float32(MARGIN))

    tile_loss = lax.fori_loop(0, bt, b_iter, jnp.float32(0.0))
    lossv[...] = jnp.broadcast_to(tile_loss, (LANES,))
    pltpu.sync_copy(lossv, out.at[wid])


def kernel(user_ids, pos_ids, neg_ids, user_table, item_table):
    batch, nneg = neg_ids.shape
    nid2 = jnp.concatenate(
        [neg_ids[:, :100], neg_ids[:, :4], neg_ids[:, 100:], neg_ids[:, 4:8]],
        axis=1).reshape(batch, 2, HALF)
    mesh = plsc.VectorSubcoreMesh(core_axis_name="c", subcore_axis_name="s")
    f = pl.kernel(
        _sc_body,
        out_type=jax.ShapeDtypeStruct((NW, LANES), jnp.float32),
        mesh=mesh,
        compiler_params=pltpu.CompilerParams(needs_layout_passes=False),
        scratch_types=[
            pltpu.VMEM((2, CH), jnp.int32),
            pltpu.VMEM((2, CH), jnp.int32),
            pltpu.VMEM((2, CH, 2, HALF), jnp.int32),
            pltpu.VMEM((2, CH, D), jnp.float32),
            pltpu.VMEM((2, CH, D), jnp.float32),
            pltpu.VMEM((2, NNEG_PAD, D), jnp.float32),
            pltpu.VMEM((LANES,), jnp.float32),
            pltpu.SemaphoreType.DMA,
            pltpu.SemaphoreType.DMA,
            pltpu.SemaphoreType.DMA,
            pltpu.SemaphoreType.DMA,
        ],
    )
    partials = f(user_ids, pos_ids, nid2, user_table, item_table)
    return jnp.sum(partials[:, 0])


# speedup vs baseline: 1.1159x; 1.0408x over previous
import jax
import jax.numpy as jnp
from jax import lax
from jax.experimental import pallas as pl
from jax.experimental.pallas import tpu as pltpu
from jax.experimental.pallas import tpu_sc as plsc

NC = 2
NS = 16
NW = NC * NS
D = 128
LANES = 16
NCH = D // LANES
CH = 32
HALF = 104
REST = 96
NNEG = HALF + REST
MARGIN = 0.5


def _sc_body(uids, pids, nids, utab, itab, out,
             uidv, pidv, nidv, urows, prows, nrows, lossv,
             semn0, semn1, semu, semp):
    wid = lax.axis_index("s") * NC + lax.axis_index("c")
    bt = uids.shape[0] // NW
    n_chunks = bt // CH
    base = wid * bt

    def stage_chunk(lc, q):
        b0 = base + lc * CH
        pltpu.sync_copy(uids.at[pl.ds(b0, CH)], uidv.at[q])
        pltpu.sync_copy(pids.at[pl.ds(b0, CH)], pidv.at[q])
        pltpu.sync_copy(nids.at[pl.ds(b0, CH)], nidv.at[q])
        pltpu.async_copy(utab.at[uidv.at[q]], urows.at[q], semu)
        pltpu.async_copy(itab.at[pidv.at[q]], prows.at[q], semp)

    def issue_neg(bl, p):
        q = (bl // CH) & 1
        bi = bl % CH
        pltpu.async_copy(itab.at[nidv.at[q, bi, 0]],
                         nrows.at[p, pl.ds(0, HALF)], semn0)
        pltpu.async_copy(itab.at[nidv.at[q, bi, 1, pl.ds(0, REST)]],
                         nrows.at[p, pl.ds(HALF, REST)], semn1)

    stage_chunk(0, 0)
    issue_neg(0, 0)

    def b_iter(bl, tl):
        p = bl & 1
        lc = bl // CH
        q = lc & 1
        bi = bl % CH

        @pl.when(bi == 0)
        def _():
            pltpu.make_async_copy(utab.at[pl.ds(0, CH)],
                                  urows.at[q], semu).wait()
            pltpu.make_async_copy(itab.at[pl.ds(0, CH)],
                                  prows.at[q], semp).wait()

        pltpu.make_async_copy(itab.at[pl.ds(0, HALF)],
                              nrows.at[p, pl.ds(0, HALF)], semn0).wait()
        pltpu.make_async_copy(itab.at[pl.ds(0, REST)],
                              nrows.at[p, pl.ds(HALF, REST)], semn1).wait()

        @pl.when((bi == CH - 1) & (lc + 1 < n_chunks))
        def _():
            stage_chunk(lc + 1, 1 - q)

        @pl.when(bl + 1 < bt)
        def _():
            issue_neg(bl + 1, 1 - p)

        u = [urows[q, bi, pl.ds(c * LANES, LANES)] for c in range(NCH)]
        accp = jnp.zeros((LANES,), jnp.float32)
        for c in range(NCH):
            dv = prows[q, bi, pl.ds(c * LANES, LANES)] - u[c]
            accp = accp + dv * dv
        pd = jnp.sum(accp)

        def neg_row(n, mn):
            acc = jnp.zeros((LANES,), jnp.float32)
            for c in range(NCH):
                dv = nrows[p, n, pl.ds(c * LANES, LANES)] - u[c]
                acc = acc + dv * dv
            return jnp.minimum(mn, jnp.sum(acc))

        mn = plsc.parallel_loop(0, NNEG, 1, unroll=4,
                                carry=jnp.float32(3.0e38))(neg_row)
        delta = mn - pd
        return tl + jnp.abs(delta - jnp.float32(MARGIN))

    tile_loss = lax.fori_loop(0, bt, b_iter, jnp.float32(0.0))
    lossv[...] = jnp.broadcast_to(tile_loss, (LANES,))
    pltpu.sync_copy(lossv, out.at[wid])


def kernel(user_ids, pos_ids, neg_ids, user_table, item_table):
    batch, nneg = neg_ids.shape
    nid2 = jnp.concatenate(
        [neg_ids[:, :HALF], neg_ids[:, HALF:], neg_ids[:, :2 * HALF - nneg]],
        axis=1).reshape(batch, 2, HALF)
    mesh = plsc.VectorSubcoreMesh(core_axis_name="c", subcore_axis_name="s")
    f = pl.kernel(
        _sc_body,
        out_type=jax.ShapeDtypeStruct((NW, LANES), jnp.float32),
        mesh=mesh,
        compiler_params=pltpu.CompilerParams(needs_layout_passes=False),
        scratch_types=[
            pltpu.VMEM((2, CH), jnp.int32),
            pltpu.VMEM((2, CH), jnp.int32),
            pltpu.VMEM((2, CH, 2, HALF), jnp.int32),
            pltpu.VMEM((2, CH, D), jnp.float32),
            pltpu.VMEM((2, CH, D), jnp.float32),
            pltpu.VMEM((2, NNEG, D), jnp.float32),
            pltpu.VMEM((LANES,), jnp.float32),
            pltpu.SemaphoreType.DMA,
            pltpu.SemaphoreType.DMA,
            pltpu.SemaphoreType.DMA,
            pltpu.SemaphoreType.DMA,
        ],
    )
    partials = f(user_ids, pos_ids, nid2, user_table, item_table)
    return jnp.sum(partials[:, 0])


# speedup vs baseline: 1.1287x; 1.0114x over previous
import jax
import jax.numpy as jnp
from jax import lax
from jax.experimental import pallas as pl
from jax.experimental.pallas import tpu as pltpu
from jax.experimental.pallas import tpu_sc as plsc

NC = 2
NS = 16
NW = NC * NS
D = 128
LANES = 16
NCH = D // LANES
CH = 64
HALF = 104
REST = 96
NNEG = HALF + REST
MARGIN = 0.5


def _sc_body(uids, pids, nids, utab, itab, out,
             uidv, pidv, nidv, urows, prows, nrows, lossv,
             semn0, semn1, semu, semp):
    wid = lax.axis_index("s") * NC + lax.axis_index("c")
    bt = uids.shape[0] // NW
    n_chunks = bt // CH
    base = wid * bt

    def stage_chunk(lc, q):
        b0 = base + lc * CH
        pltpu.sync_copy(uids.at[pl.ds(b0, CH)], uidv.at[q])
        pltpu.sync_copy(pids.at[pl.ds(b0, CH)], pidv.at[q])
        pltpu.sync_copy(nids.at[pl.ds(b0, CH)], nidv.at[q])
        pltpu.async_copy(utab.at[uidv.at[q]], urows.at[q], semu)
        pltpu.async_copy(itab.at[pidv.at[q]], prows.at[q], semp)

    def issue_neg(bl, p):
        q = (bl // CH) & 1
        bi = bl % CH
        pltpu.async_copy(itab.at[nidv.at[q, bi, 0]],
                         nrows.at[p, pl.ds(0, HALF)], semn0)
        pltpu.async_copy(itab.at[nidv.at[q, bi, 1, pl.ds(0, REST)]],
                         nrows.at[p, pl.ds(HALF, REST)], semn1)

    stage_chunk(0, 0)
    issue_neg(0, 0)

    def b_iter(bl, tl):
        p = bl & 1
        lc = bl // CH
        q = lc & 1
        bi = bl % CH

        @pl.when(bi == 0)
        def _():
            pltpu.make_async_copy(utab.at[pl.ds(0, CH)],
                                  urows.at[q], semu).wait()
            pltpu.make_async_copy(itab.at[pl.ds(0, CH)],
                                  prows.at[q], semp).wait()

        pltpu.make_async_copy(itab.at[pl.ds(0, HALF)],
                              nrows.at[p, pl.ds(0, HALF)], semn0).wait()
        pltpu.make_async_copy(itab.at[pl.ds(0, REST)],
                              nrows.at[p, pl.ds(HALF, REST)], semn1).wait()

        @pl.when((bi == CH - 1) & (lc + 1 < n_chunks))
        def _():
            stage_chunk(lc + 1, 1 - q)

        @pl.when(bl + 1 < bt)
        def _():
            issue_neg(bl + 1, 1 - p)

        u = [urows[q, bi, pl.ds(c * LANES, LANES)] for c in range(NCH)]
        accp = jnp.zeros((LANES,), jnp.float32)
        for c in range(NCH):
            dv = prows[q, bi, pl.ds(c * LANES, LANES)] - u[c]
            accp = accp + dv * dv
        pd = jnp.sum(accp)

        def neg_row(n, mn):
            acc = jnp.zeros((LANES,), jnp.float32)
            for c in range(NCH):
                dv = nrows[p, n, pl.ds(c * LANES, LANES)] - u[c]
                acc = acc + dv * dv
            return jnp.minimum(mn, jnp.sum(acc))

        mn = plsc.parallel_loop(0, NNEG, 1, unroll=4,
                                carry=jnp.float32(3.0e38))(neg_row)
        delta = mn - pd
        return tl + jnp.abs(delta - jnp.float32(MARGIN))

    tile_loss = lax.fori_loop(0, bt, b_iter, jnp.float32(0.0))
    lossv[...] = jnp.broadcast_to(tile_loss, (LANES,))
    pltpu.sync_copy(lossv, out.at[wid])


def kernel(user_ids, pos_ids, neg_ids, user_table, item_table):
    batch, nneg = neg_ids.shape
    nid2 = jnp.concatenate(
        [neg_ids[:, :HALF], neg_ids[:, HALF:], neg_ids[:, :2 * HALF - nneg]],
        axis=1).reshape(batch, 2, HALF)
    mesh = plsc.VectorSubcoreMesh(core_axis_name="c", subcore_axis_name="s")
    f = pl.kernel(
        _sc_body,
        out_type=jax.ShapeDtypeStruct((NW, LANES), jnp.float32),
        mesh=mesh,
        compiler_params=pltpu.CompilerParams(needs_layout_passes=False),
        scratch_types=[
            pltpu.VMEM((2, CH), jnp.int32),
            pltpu.VMEM((2, CH), jnp.int32),
            pltpu.VMEM((2, CH, 2, HALF), jnp.int32),
            pltpu.VMEM((2, CH, D), jnp.float32),
            pltpu.VMEM((2, CH, D), jnp.float32),
            pltpu.VMEM((2, NNEG, D), jnp.float32),
            pltpu.VMEM((LANES,), jnp.float32),
            pltpu.SemaphoreType.DMA,
            pltpu.SemaphoreType.DMA,
            pltpu.SemaphoreType.DMA,
            pltpu.SemaphoreType.DMA,
        ],
    )
    partials = f(user_ids, pos_ids, nid2, user_table, item_table)
    return jnp.sum(partials[:, 0])
